# baseline (speedup 1.0000x reference)
import functools

import jax
import jax.numpy as jnp
from jax import lax
from jax.experimental import pallas as pl
from jax.experimental.pallas import tpu as pltpu
from jax.experimental.pallas import tpu_sc as plsc

_B, _P, _D = 64, 1024, 768
_NC, _NS, _L = 2, 16, 16
_NW = _NC * _NS
_PW = _P // _NW
_NCHUNK = _D // _L
_NBUF = 4


def _compute(x_v, pos_v):
    def row_body(r, carry):
        for c in range(_NCHUNK):
            sl = pl.ds(c * _L, _L)
            plsc.addupdate(x_v.at[r, sl], pos_v[r, sl])
        return carry

    lax.fori_loop(0, _PW, row_body, 0)


def _sc_kernel_body(x_hbm, pos_hbm, out_hbm, pos_v, bufs, in_sems, out_sems):
    wid = lax.axis_index("s") * _NC + lax.axis_index("c")
    psl = pl.ds(wid * _PW, _PW)

    def step(b, k, first, last):
        k2 = (k + 2) % _NBUF
        pltpu.make_async_copy(x_hbm.at[b, psl], bufs[k], in_sems[k]).wait()
        _compute(bufs[k], pos_v)
        pltpu.async_copy(bufs[k], out_hbm.at[b, psl], out_sems[k])
        if not last:
            if not first:
                pltpu.make_async_copy(
                    bufs[k2], out_hbm.at[b - 2, psl], out_sems[k2]).wait()
            pltpu.async_copy(x_hbm.at[b + 2, psl], bufs[k2], in_sems[k2])

    pltpu.sync_copy(pos_hbm.at[psl], pos_v)
    pltpu.async_copy(x_hbm.at[0, psl], bufs[0], in_sems[0])
    pltpu.async_copy(x_hbm.at[1, psl], bufs[1], in_sems[1])

    step(0, 0, first=True, last=False)
    step(1, 1, first=True, last=False)

    def group_body(i, carry):
        b0 = 4 * i + 2
        step(b0 + 0, 2, first=False, last=False)
        step(b0 + 1, 3, first=False, last=False)
        step(b0 + 2, 0, first=False, last=False)
        step(b0 + 3, 1, first=False, last=False)
        return carry

    lax.fori_loop(0, (_B - 4) // 4, group_body, 0)

    step(_B - 2, 2, first=False, last=True)
    step(_B - 1, 3, first=False, last=True)

    for b in range(_B - 4, _B):
        k = b % _NBUF
        pltpu.make_async_copy(bufs[k], out_hbm.at[b, psl], out_sems[k]).wait()


@functools.partial(
    pl.kernel,
    out_type=jax.ShapeDtypeStruct((_B, _P, _D), jnp.float32),
    mesh=plsc.VectorSubcoreMesh(
        core_axis_name="c", subcore_axis_name="s",
        num_cores=_NC, num_subcores=_NS,
    ),
    scratch_types=[
        pltpu.VMEM((_PW, _D), jnp.float32),
        pltpu.VMEM((_PW, _D), jnp.float32),
        pltpu.VMEM((_PW, _D), jnp.float32),
        pltpu.VMEM((_PW, _D), jnp.float32),
        pltpu.VMEM((_PW, _D), jnp.float32),
        pltpu.SemaphoreType.DMA,
        pltpu.SemaphoreType.DMA,
        pltpu.SemaphoreType.DMA,
        pltpu.SemaphoreType.DMA,
        pltpu.SemaphoreType.DMA,
        pltpu.SemaphoreType.DMA,
        pltpu.SemaphoreType.DMA,
        pltpu.SemaphoreType.DMA,
    ],
)
def _sc_kernel(x_hbm, pos_hbm, out_hbm, pos_v, b0, b1, b2, b3,
               is0, is1, is2, is3, os0, os1, os2, os3):
    _sc_kernel_body(x_hbm, pos_hbm, out_hbm, pos_v,
                    (b0, b1, b2, b3), (is0, is1, is2, is3),
                    (os0, os1, os2, os3))


def kernel(encoded_patches, pos_table):
    return _sc_kernel(encoded_patches, pos_table)
